# baseline (speedup 1.0000x reference)
import functools

import jax
import jax.numpy as jnp
from jax import lax
from jax.experimental import pallas as pl
from jax.experimental.pallas import tpu as pltpu
from jax.experimental.pallas import tpu_sc as plsc

N = 10000
E = 320000
B = 128
D = 128

NC = 2
NS = 16
DW = D // NC

CNTW = 8

IDXROWS = E // 128
CHUNK_IDXROWS = 2
CHUNK_EDGES = CHUNK_IDXROWS * 128
NCHUNKS = IDXROWS // CHUNK_IDXROWS
CHUNKS_PER_TILE = NCHUNKS // NS
EXTRA_CHUNKS = NCHUNKS - CHUNKS_PER_TILE * NS
TILE_IDXROWS = CHUNKS_PER_TILE * CHUNK_IDXROWS
ROWS_PER_TILE = N // NS

NODE_BLK = 1000
NB = N // NODE_BLK


NSLOT = 3
MAIN_CHUNKS = (CHUNKS_PER_TILE // NSLOT) * NSLOT
TAIL_CHUNKS = CHUNKS_PER_TILE - MAIN_CHUNKS


def _sc_body(ea_hbm, idx_hbm, zrow_hbm, zcnt_hbm, ones_hbm,
             esum_out, cnt_out,
             rows, idx_v, ones_v, acc, cacc, gsem, ssem):
    c = lax.axis_index("c")
    s = lax.axis_index("s")
    col0 = c * DW

    base_n = s * ROWS_PER_TILE
    pltpu.sync_copy(zrow_hbm, acc.at[pl.ds(base_n, ROWS_PER_TILE)])
    pltpu.sync_copy(zcnt_hbm, cacc.at[pl.ds(base_n, ROWS_PER_TILE)])
    pltpu.sync_copy(ones_hbm, ones_v)
    pltpu.sync_copy(idx_hbm.at[pl.ds(s * TILE_IDXROWS, TILE_IDXROWS)],
                    idx_v.at[pl.ds(0, TILE_IDXROWS)])

    @pl.when(s < EXTRA_CHUNKS)
    def _():
        pltpu.sync_copy(
            idx_hbm.at[pl.ds(NS * TILE_IDXROWS + CHUNK_IDXROWS * s,
                             CHUNK_IDXROWS)],
            idx_v.at[pl.ds(TILE_IDXROWS, CHUNK_IDXROWS)])

    plsc.subcore_barrier()

    def src(k):
        return ea_hbm.at[pl.ds(k * CHUNK_EDGES, CHUNK_EDGES), pl.ds(col0, DW)]

    def gather_start(k, b):
        pltpu.async_copy(src(k), rows[b], gsem[b])

    def gather_wait(b):
        pltpu.make_async_copy(src(0), rows[b], gsem[b]).wait()

    def scatter_start(b, r, count_this):
        for j in range(CHUNK_IDXROWS):
            pltpu.async_copy(rows[b].at[pl.ds(j * 128, 128)],
                             acc.at[idx_v.at[r + j]], ssem[b], add=True)

        @pl.when(count_this)
        def _():
            for j in range(CHUNK_IDXROWS):
                pltpu.async_copy(ones_v, cacc.at[idx_v.at[r + j]],
                                 ssem[b], add=True)

    def scatter_drain(b, count_this):
        for j in range(CHUNK_IDXROWS):
            pltpu.make_async_copy(rows[b].at[pl.ds(j * 128, 128)],
                                  acc.at[idx_v.at[0]], ssem[b]).wait()

        @pl.when(count_this)
        def _():
            for j in range(CHUNK_IDXROWS):
                pltpu.make_async_copy(ones_v, cacc.at[idx_v.at[0]],
                                      ssem[b]).wait()

    def owner(k_parity):
        return c == k_parity

    base = s * CHUNKS_PER_TILE
    for b in range(NSLOT - 1):
        gather_start(base + b, b)

    def pipe_step(k0, b, drain_prev):
        rel = k0 - base + b
        gather_wait(b)
        scatter_start(b, 2 * rel, owner(lax.rem(rel, 2)))
        nb = (b + NSLOT - 1) % NSLOT
        if drain_prev:
            scatter_drain(nb, owner(lax.rem(rel + 1, 2)))
        gather_start(k0 + b + NSLOT - 1, nb)

    for b in range(NSLOT):
        pipe_step(base, b, drain_prev=(b != 0))

    def loop_body(mm, carry):
        k0 = base + NSLOT * mm
        for b in range(NSLOT):
            pipe_step(k0, b, drain_prev=True)
        return carry

    lax.fori_loop(1, MAIN_CHUNKS // NSLOT, loop_body, 0)

    for t in range(TAIL_CHUNKS):
        gather_wait(t)
        scatter_start(t, 2 * (MAIN_CHUNKS + t), owner(t % 2))
    for g in range(TAIL_CHUNKS, NSLOT - 1):
        gather_wait(g)
    scatter_drain(NSLOT - 1, owner((MAIN_CHUNKS - 1) % 2))
    for t in range(TAIL_CHUNKS):
        scatter_drain(t, owner(t % 2))

    @pl.when(s < EXTRA_CHUNKS)
    def _():
        k = NS * CHUNKS_PER_TILE + s
        pltpu.sync_copy(src(k), rows[NSLOT - 1])
        scatter_start(NSLOT - 1, TILE_IDXROWS, c == (s % 2))
        scatter_drain(NSLOT - 1, c == (s % 2))

    plsc.subcore_barrier()
    pltpu.sync_copy(acc.at[pl.ds(base_n, ROWS_PER_TILE)],
                    esum_out.at[pl.ds(base_n, ROWS_PER_TILE), pl.ds(col0, DW)])
    pltpu.sync_copy(cacc.at[pl.ds(base_n, ROWS_PER_TILE)],
                    cnt_out.at[c, pl.ds(base_n, ROWS_PER_TILE)])


_sc_scatter = functools.partial(
    pl.kernel,
    out_type=(
        jax.ShapeDtypeStruct((N, D), jnp.float32),
        jax.ShapeDtypeStruct((NC, N, CNTW), jnp.float32),
    ),
    mesh=plsc.VectorSubcoreMesh(
        core_axis_name="c", subcore_axis_name="s",
        num_cores=NC, num_subcores=NS),
    scratch_types=[
        [pltpu.VMEM((CHUNK_EDGES, DW), jnp.float32) for _ in range(NSLOT)],
        pltpu.VMEM((TILE_IDXROWS + CHUNK_IDXROWS, 128), jnp.int32),
        pltpu.VMEM((128, CNTW), jnp.float32),
        pltpu.VMEM_SHARED((N, DW), jnp.float32),
        pltpu.VMEM_SHARED((N, CNTW), jnp.float32),
        [pltpu.SemaphoreType.DMA for _ in range(NSLOT)],
        [pltpu.SemaphoreType.DMA for _ in range(NSLOT)],
    ],
    compiler_params=pltpu.CompilerParams(use_tc_tiling_on_sc=False),
)(_sc_body)


def _tcx_body(x_ref, batch_ref, accx_out, accn_out, acc_x, acc_n):
    j = pl.program_id(0)

    @pl.when(j == 0)
    def _():
        acc_x[...] = jnp.zeros((B, D), jnp.float32)
        acc_n[...] = jnp.zeros((B, D), jnp.float32)

    bb = batch_ref[0]
    ids = lax.broadcasted_iota(jnp.int32, (B, NODE_BLK), 0)
    m = (bb == ids).astype(jnp.float32)

    acc_x[...] += jnp.dot(m, x_ref[...], preferred_element_type=jnp.float32)
    acc_n[...] += jnp.sum(m, axis=1, keepdims=True)

    @pl.when(j == NB - 1)
    def _():
        accx_out[...] = acc_x[...]
        accn_out[...] = acc_n[...]


def _tcx_call(x, batch3):
    res = lambda j: (0, 0)
    return pl.pallas_call(
        _tcx_body,
        grid=(NB,),
        in_specs=[
            pl.BlockSpec((NODE_BLK, D), lambda j: (j, 0)),
            pl.BlockSpec((1, 1, NODE_BLK), lambda j: (j, 0, 0)),
        ],
        out_specs=[pl.BlockSpec((B, D), res), pl.BlockSpec((B, D), res)],
        out_shape=[jax.ShapeDtypeStruct((B, D), jnp.float32),
                   jax.ShapeDtypeStruct((B, D), jnp.float32)],
        scratch_shapes=[pltpu.VMEM((B, D), jnp.float32)] * 2,
        compiler_params=pltpu.CompilerParams(
            dimension_semantics=("arbitrary",)),
    )(x, batch3)


def _tce_body(esum_hbm, cnt_ref, batch_ref, accx_ref, accn_ref, u_ref,
              w0_ref, w1_ref, w2_ref, b0_ref, b1_ref, b2_ref,
              out_ref, acc_e, ebuf, esem):
    j = pl.program_id(0)

    @pl.when(j == 0)
    def _():
        acc_e[...] = jnp.zeros((B, D), jnp.float32)
        pltpu.async_copy(esum_hbm, ebuf, esem).wait()

    bb = batch_ref[0]
    ids = lax.broadcasted_iota(jnp.int32, (B, NODE_BLK), 0)
    m = (bb == ids).astype(jnp.float32)

    deg = cnt_ref[0, :, 0:1] + cnt_ref[1, :, 0:1]
    inv = 1.0 / jnp.maximum(deg, 1.0)
    row0 = pl.multiple_of(j * NODE_BLK, 8)
    e = ebuf[pl.ds(row0, NODE_BLK), :] * inv

    acc_e[...] += jnp.dot(m, e, preferred_element_type=jnp.float32)

    @pl.when(j == NB - 1)
    def _():
        n = jnp.maximum(accn_ref[...], 1.0)
        u_e = acc_e[...] / n
        u_v = accx_ref[...] / n
        comb = jnp.concatenate([u_e, u_v, u_ref[...]], axis=1)
        dn = (((1,), (1,)), ((), ()))
        h = jnp.maximum(lax.dot_general(
            comb, w0_ref[...], dn, preferred_element_type=jnp.float32)
            + b0_ref[...], 0.0)
        h = jnp.maximum(lax.dot_general(
            h, w1_ref[...], dn, preferred_element_type=jnp.float32)
            + b1_ref[...], 0.0)
        h = jnp.maximum(lax.dot_general(
            h, w2_ref[...], dn, preferred_element_type=jnp.float32)
            + b2_ref[...], 0.0)
        out_ref[...] = h


def _tce_call(esum, cnt, batch3, accx, accn, u, W0, W1, W2, b0r, b1r, b2r):
    res = lambda j: (0, 0)
    return pl.pallas_call(
        _tce_body,
        grid=(NB,),
        in_specs=[
            pl.BlockSpec(memory_space=pl.ANY),
            pl.BlockSpec((NC, NODE_BLK, CNTW), lambda j: (0, j, 0)),
            pl.BlockSpec((1, 1, NODE_BLK), lambda j: (j, 0, 0)),
            pl.BlockSpec((B, D), res),
            pl.BlockSpec((B, D), res),
            pl.BlockSpec((B, D), res),
            pl.BlockSpec((D, 3 * D), res),
            pl.BlockSpec((D, D), res),
            pl.BlockSpec((D, D), res),
            pl.BlockSpec((1, D), res),
            pl.BlockSpec((1, D), res),
            pl.BlockSpec((1, D), res),
        ],
        out_specs=pl.BlockSpec((B, D), res),
        out_shape=jax.ShapeDtypeStruct((B, D), jnp.float32),
        scratch_shapes=[pltpu.VMEM((B, D), jnp.float32),
                        pltpu.VMEM((N, D), jnp.float32),
                        pltpu.SemaphoreType.DMA],
        compiler_params=pltpu.CompilerParams(
            dimension_semantics=("arbitrary",)),
    )(esum, cnt, batch3, accx, accn, u, W0, W1, W2, b0r, b1r, b2r)


def kernel(x, edge_index, edge_attr, u, batch, W0, b0, W1, b1, W2, b2):
    assert x.shape == (N, D) and edge_attr.shape == (E, D)
    assert u.shape == (B, D) and batch.shape == (N,)

    src = edge_index[0].reshape(IDXROWS, 128)
    zrow = jnp.zeros((ROWS_PER_TILE, DW), jnp.float32)
    zcnt = jnp.zeros((ROWS_PER_TILE, CNTW), jnp.float32)
    ones16 = jnp.ones((128, CNTW), jnp.float32)

    esum, cnt = _sc_scatter(edge_attr, src, zrow, zcnt, ones16)

    batch3 = batch.reshape(NB, 1, NODE_BLK)
    accx, accn = _tcx_call(x, batch3)
    return _tce_call(esum, cnt, batch3, accx, accn, u, W0, W1, W2,
                     b0.reshape(1, D), b1.reshape(1, D), b2.reshape(1, D))


# speedup vs baseline: 1.0026x; 1.0026x over previous
import functools

import jax
import jax.numpy as jnp
from jax import lax
from jax.experimental import pallas as pl
from jax.experimental.pallas import tpu as pltpu
from jax.experimental.pallas import tpu_sc as plsc

N = 10000
E = 320000
B = 128
D = 128

NC = 2
NS = 16
DW = D // NC

CNTW = 8

IDXROWS = E // 128
CHUNK_IDXROWS = 2
CHUNK_EDGES = CHUNK_IDXROWS * 128
NCHUNKS = IDXROWS // CHUNK_IDXROWS
CHUNKS_PER_TILE = NCHUNKS // NS
EXTRA_CHUNKS = NCHUNKS - CHUNKS_PER_TILE * NS
TILE_IDXROWS = CHUNKS_PER_TILE * CHUNK_IDXROWS
ROWS_PER_TILE = N // NS

NODE_BLK = 1000
NB = N // NODE_BLK


NSLOT = 3
MAIN_CHUNKS = (CHUNKS_PER_TILE // NSLOT) * NSLOT
TAIL_CHUNKS = CHUNKS_PER_TILE - MAIN_CHUNKS


def _sc_body(ea_hbm, idx_hbm, zrow_hbm, zcnt_hbm, ones_hbm,
             esum_out, cnt_out,
             rows, idx_v, ones_v, acc, cacc, gsem, ssem):
    c = lax.axis_index("c")
    s = lax.axis_index("s")
    col0 = c * DW

    base_n = s * ROWS_PER_TILE
    pltpu.sync_copy(zrow_hbm, acc.at[pl.ds(base_n, ROWS_PER_TILE)])
    pltpu.sync_copy(zcnt_hbm, cacc.at[pl.ds(base_n, ROWS_PER_TILE)])
    pltpu.sync_copy(ones_hbm, ones_v)
    pltpu.sync_copy(idx_hbm.at[pl.ds(s * TILE_IDXROWS, TILE_IDXROWS)],
                    idx_v.at[pl.ds(0, TILE_IDXROWS)])

    @pl.when(s < EXTRA_CHUNKS)
    def _():
        pltpu.sync_copy(
            idx_hbm.at[pl.ds(NS * TILE_IDXROWS + CHUNK_IDXROWS * s,
                             CHUNK_IDXROWS)],
            idx_v.at[pl.ds(TILE_IDXROWS, CHUNK_IDXROWS)])

    plsc.subcore_barrier()

    def src(k):
        return ea_hbm.at[pl.ds(k * CHUNK_EDGES, CHUNK_EDGES), pl.ds(col0, DW)]

    def gather_start(k, b):
        pltpu.async_copy(src(k), rows[b], gsem[b])

    def gather_wait(b):
        pltpu.make_async_copy(src(0), rows[b], gsem[b]).wait()

    def scatter_start(b, r, count_this):
        for j in range(CHUNK_IDXROWS):
            pltpu.async_copy(rows[b].at[pl.ds(j * 128, 128)],
                             acc.at[idx_v.at[r + j]], ssem[b], add=True)

        @pl.when(count_this)
        def _():
            for j in range(CHUNK_IDXROWS):
                pltpu.async_copy(ones_v, cacc.at[idx_v.at[r + j]],
                                 ssem[b], add=True)

    def scatter_drain(b, count_this):
        for j in range(CHUNK_IDXROWS):
            pltpu.make_async_copy(rows[b].at[pl.ds(j * 128, 128)],
                                  acc.at[idx_v.at[0]], ssem[b]).wait()

        @pl.when(count_this)
        def _():
            for j in range(CHUNK_IDXROWS):
                pltpu.make_async_copy(ones_v, cacc.at[idx_v.at[0]],
                                      ssem[b]).wait()

    def owner(k_parity):
        return c == k_parity

    base = s * CHUNKS_PER_TILE
    for b in range(NSLOT - 1):
        gather_start(base + b, b)

    def pipe_step(k0, b, drain_prev):
        rel = k0 - base + b
        gather_wait(b)
        scatter_start(b, 2 * rel, owner(lax.rem(rel, 2)))
        nb = (b + NSLOT - 1) % NSLOT
        if drain_prev:
            scatter_drain(nb, owner(lax.rem(rel + 1, 2)))
        gather_start(k0 + b + NSLOT - 1, nb)

    for b in range(NSLOT):
        pipe_step(base, b, drain_prev=(b != 0))

    def loop_body(mm, carry):
        k0 = base + NSLOT * mm
        for b in range(NSLOT):
            pipe_step(k0, b, drain_prev=True)
        return carry

    lax.fori_loop(1, MAIN_CHUNKS // NSLOT, loop_body, 0)

    for t in range(TAIL_CHUNKS):
        gather_wait(t)
        scatter_start(t, 2 * (MAIN_CHUNKS + t), owner(t % 2))
    for g in range(TAIL_CHUNKS, NSLOT - 1):
        gather_wait(g)
    scatter_drain(NSLOT - 1, owner((MAIN_CHUNKS - 1) % 2))
    for t in range(TAIL_CHUNKS):
        scatter_drain(t, owner(t % 2))

    @pl.when(s < EXTRA_CHUNKS)
    def _():
        k = NS * CHUNKS_PER_TILE + s
        pltpu.sync_copy(src(k), rows[NSLOT - 1])
        scatter_start(NSLOT - 1, TILE_IDXROWS, c == (s % 2))
        scatter_drain(NSLOT - 1, c == (s % 2))

    plsc.subcore_barrier()
    pltpu.sync_copy(acc.at[pl.ds(base_n, ROWS_PER_TILE)],
                    esum_out.at[pl.ds(base_n, ROWS_PER_TILE), pl.ds(col0, DW)])
    pltpu.sync_copy(cacc.at[pl.ds(base_n, ROWS_PER_TILE)],
                    cnt_out.at[c, pl.ds(base_n, ROWS_PER_TILE)])


_sc_scatter = functools.partial(
    pl.kernel,
    out_type=(
        jax.ShapeDtypeStruct((N, D), jnp.float32),
        jax.ShapeDtypeStruct((NC, N, CNTW), jnp.float32),
    ),
    mesh=plsc.VectorSubcoreMesh(
        core_axis_name="c", subcore_axis_name="s",
        num_cores=NC, num_subcores=NS),
    scratch_types=[
        [pltpu.VMEM((CHUNK_EDGES, DW), jnp.float32) for _ in range(NSLOT)],
        pltpu.VMEM((TILE_IDXROWS + CHUNK_IDXROWS, 128), jnp.int32),
        pltpu.VMEM((128, CNTW), jnp.float32),
        pltpu.VMEM_SHARED((N, DW), jnp.float32),
        pltpu.VMEM_SHARED((N, CNTW), jnp.float32),
        [pltpu.SemaphoreType.DMA for _ in range(NSLOT)],
        [pltpu.SemaphoreType.DMA for _ in range(NSLOT)],
    ],
    compiler_params=pltpu.CompilerParams(use_tc_tiling_on_sc=False),
)(_sc_body)


def _tcx_body(x_ref, batch_ref, accx_out, accn_out, acc_x, acc_n):
    j = pl.program_id(0)

    @pl.when(j == 0)
    def _():
        acc_x[...] = jnp.zeros((B, D), jnp.float32)
        acc_n[...] = jnp.zeros((B, D), jnp.float32)

    bb = batch_ref[0]
    ids = lax.broadcasted_iota(jnp.int32, (B, NODE_BLK), 0)
    m = (bb == ids).astype(jnp.float32)

    acc_x[...] += jnp.dot(m, x_ref[...], preferred_element_type=jnp.float32)
    acc_n[...] += jnp.sum(m, axis=1, keepdims=True)

    @pl.when(j == NB - 1)
    def _():
        accx_out[...] = acc_x[...]
        accn_out[...] = acc_n[...]


def _tcx_call(x, batch3):
    res = lambda j: (0, 0)
    return pl.pallas_call(
        _tcx_body,
        grid=(NB,),
        in_specs=[
            pl.BlockSpec((NODE_BLK, D), lambda j: (j, 0)),
            pl.BlockSpec((1, 1, NODE_BLK), lambda j: (j, 0, 0)),
        ],
        out_specs=[pl.BlockSpec((B, D), res), pl.BlockSpec((B, D), res)],
        out_shape=[jax.ShapeDtypeStruct((B, D), jnp.float32),
                   jax.ShapeDtypeStruct((B, D), jnp.float32)],
        scratch_shapes=[pltpu.VMEM((B, D), jnp.float32)] * 2,
        compiler_params=pltpu.CompilerParams(
            dimension_semantics=("arbitrary",)),
    )(x, batch3)


def _tce_body(esum_hbm, cnt_hbm, batch_ref, accx_ref, accn_ref, u_ref,
              w0_ref, w1_ref, w2_ref, b0_ref, b1_ref, b2_ref,
              out_ref, acc_e, ebuf, cbuf, esem, csem):
    j = pl.program_id(0)

    @pl.when(j == 0)
    def _():
        acc_e[...] = jnp.zeros((B, D), jnp.float32)
        ch = pltpu.async_copy(cnt_hbm, cbuf, csem)
        eh = pltpu.async_copy(esum_hbm, ebuf, esem)
        ch.wait()
        eh.wait()

    bb = batch_ref[0]
    ids = lax.broadcasted_iota(jnp.int32, (B, NODE_BLK), 0)
    m = (bb == ids).astype(jnp.float32)

    row0 = pl.multiple_of(j * NODE_BLK, 8)
    deg = cbuf[0, pl.ds(row0, NODE_BLK), 0:1] + cbuf[1, pl.ds(row0, NODE_BLK), 0:1]
    inv = 1.0 / jnp.maximum(deg, 1.0)
    e = ebuf[pl.ds(row0, NODE_BLK), :] * inv

    acc_e[...] += jnp.dot(m, e, preferred_element_type=jnp.float32)

    @pl.when(j == NB - 1)
    def _():
        n = jnp.maximum(accn_ref[...], 1.0)
        u_e = acc_e[...] / n
        u_v = accx_ref[...] / n
        comb = jnp.concatenate([u_e, u_v, u_ref[...]], axis=1)
        dn = (((1,), (1,)), ((), ()))
        h = jnp.maximum(lax.dot_general(
            comb, w0_ref[...], dn, preferred_element_type=jnp.float32)
            + b0_ref[...], 0.0)
        h = jnp.maximum(lax.dot_general(
            h, w1_ref[...], dn, preferred_element_type=jnp.float32)
            + b1_ref[...], 0.0)
        h = jnp.maximum(lax.dot_general(
            h, w2_ref[...], dn, preferred_element_type=jnp.float32)
            + b2_ref[...], 0.0)
        out_ref[...] = h


def _tce_call(esum, cnt, batch3, accx, accn, u, W0, W1, W2, b0r, b1r, b2r):
    res = lambda j: (0, 0)
    return pl.pallas_call(
        _tce_body,
        grid=(NB,),
        in_specs=[
            pl.BlockSpec(memory_space=pl.ANY),
            pl.BlockSpec(memory_space=pl.ANY),
            pl.BlockSpec((1, 1, NODE_BLK), lambda j: (j, 0, 0)),
            pl.BlockSpec((B, D), res),
            pl.BlockSpec((B, D), res),
            pl.BlockSpec((B, D), res),
            pl.BlockSpec((D, 3 * D), res),
            pl.BlockSpec((D, D), res),
            pl.BlockSpec((D, D), res),
            pl.BlockSpec((1, D), res),
            pl.BlockSpec((1, D), res),
            pl.BlockSpec((1, D), res),
        ],
        out_specs=pl.BlockSpec((B, D), res),
        out_shape=jax.ShapeDtypeStruct((B, D), jnp.float32),
        scratch_shapes=[pltpu.VMEM((B, D), jnp.float32),
                        pltpu.VMEM((N, D), jnp.float32),
                        pltpu.VMEM((NC, N, CNTW), jnp.float32),
                        pltpu.SemaphoreType.DMA,
                        pltpu.SemaphoreType.DMA],
        compiler_params=pltpu.CompilerParams(
            dimension_semantics=("arbitrary",)),
    )(esum, cnt, batch3, accx, accn, u, W0, W1, W2, b0r, b1r, b2r)


def kernel(x, edge_index, edge_attr, u, batch, W0, b0, W1, b1, W2, b2):
    assert x.shape == (N, D) and edge_attr.shape == (E, D)
    assert u.shape == (B, D) and batch.shape == (N,)

    src = edge_index[0].reshape(IDXROWS, 128)
    zrow = jnp.zeros((ROWS_PER_TILE, DW), jnp.float32)
    zcnt = jnp.zeros((ROWS_PER_TILE, CNTW), jnp.float32)
    ones16 = jnp.ones((128, CNTW), jnp.float32)

    esum, cnt = _sc_scatter(edge_attr, src, zrow, zcnt, ones16)

    batch3 = batch.reshape(NB, 1, NODE_BLK)
    accx, accn = _tcx_call(x, batch3)
    return _tce_call(esum, cnt, batch3, accx, accn, u, W0, W1, W2,
                     b0.reshape(1, D), b1.reshape(1, D), b2.reshape(1, D))


# speedup vs baseline: 1.0632x; 1.0604x over previous
import functools

import jax
import jax.numpy as jnp
from jax import lax
from jax.experimental import pallas as pl
from jax.experimental.pallas import tpu as pltpu
from jax.experimental.pallas import tpu_sc as plsc

N = 10000
E = 320000
B = 128
D = 128

NC = 2
NS = 16
DW = D // NC

CNTW = 8

IDXROWS = E // 128
CHUNK_IDXROWS = 2
CHUNK_EDGES = CHUNK_IDXROWS * 128
NCHUNKS = IDXROWS // CHUNK_IDXROWS
CHUNKS_PER_TILE = NCHUNKS // NS
EXTRA_CHUNKS = NCHUNKS - CHUNKS_PER_TILE * NS
TILE_IDXROWS = CHUNKS_PER_TILE * CHUNK_IDXROWS
ROWS_PER_TILE = N // NS

NODE_BLK = 1000
NB = N // NODE_BLK


CGROUPS = (ROWS_PER_TILE + 15) // 16

NSLOT = 3
MAIN_CHUNKS = (CHUNKS_PER_TILE // NSLOT) * NSLOT
TAIL_CHUNKS = CHUNKS_PER_TILE - MAIN_CHUNKS


def _sc_body(ea_hbm, idx_hbm, zrow_hbm, zcnt_hbm, ones_hbm,
             esum_out, cnt_out,
             rows, idx_v, ones_v, cc_v, cflat_v, acc, cacc, gsem, ssem):
    c = lax.axis_index("c")
    s = lax.axis_index("s")
    col0 = c * DW

    base_n = s * ROWS_PER_TILE
    pltpu.sync_copy(zrow_hbm, acc.at[pl.ds(base_n, ROWS_PER_TILE)])
    pltpu.sync_copy(zcnt_hbm, cacc.at[pl.ds(base_n, ROWS_PER_TILE)])
    pltpu.sync_copy(ones_hbm, ones_v)
    pltpu.sync_copy(idx_hbm.at[pl.ds(s * TILE_IDXROWS, TILE_IDXROWS)],
                    idx_v.at[pl.ds(0, TILE_IDXROWS)])

    @pl.when(s < EXTRA_CHUNKS)
    def _():
        pltpu.sync_copy(
            idx_hbm.at[pl.ds(NS * TILE_IDXROWS + CHUNK_IDXROWS * s,
                             CHUNK_IDXROWS)],
            idx_v.at[pl.ds(TILE_IDXROWS, CHUNK_IDXROWS)])

    plsc.subcore_barrier()

    def src(k):
        return ea_hbm.at[pl.ds(k * CHUNK_EDGES, CHUNK_EDGES), pl.ds(col0, DW)]

    def gather_start(k, b):
        pltpu.async_copy(src(k), rows[b], gsem[b])

    def gather_wait(b):
        pltpu.make_async_copy(src(0), rows[b], gsem[b]).wait()

    def scatter_start(b, r, count_this):
        for j in range(CHUNK_IDXROWS):
            pltpu.async_copy(rows[b].at[pl.ds(j * 128, 128)],
                             acc.at[idx_v.at[r + j]], ssem[b], add=True)

        @pl.when(count_this)
        def _():
            for j in range(CHUNK_IDXROWS):
                pltpu.async_copy(ones_v, cacc.at[idx_v.at[r + j]],
                                 ssem[b], add=True)

    def scatter_drain(b, count_this):
        for j in range(CHUNK_IDXROWS):
            pltpu.make_async_copy(rows[b].at[pl.ds(j * 128, 128)],
                                  acc.at[idx_v.at[0]], ssem[b]).wait()

        @pl.when(count_this)
        def _():
            for j in range(CHUNK_IDXROWS):
                pltpu.make_async_copy(ones_v, cacc.at[idx_v.at[0]],
                                      ssem[b]).wait()

    def owner(k_parity):
        return c == k_parity

    base = s * CHUNKS_PER_TILE
    for b in range(NSLOT - 1):
        gather_start(base + b, b)

    def pipe_step(k0, b, drain_prev):
        rel = k0 - base + b
        gather_wait(b)
        scatter_start(b, 2 * rel, owner(lax.rem(rel, 2)))
        nb = (b + NSLOT - 1) % NSLOT
        if drain_prev:
            scatter_drain(nb, owner(lax.rem(rel + 1, 2)))
        gather_start(k0 + b + NSLOT - 1, nb)

    for b in range(NSLOT):
        pipe_step(base, b, drain_prev=(b != 0))

    def loop_body(mm, carry):
        k0 = base + NSLOT * mm
        for b in range(NSLOT):
            pipe_step(k0, b, drain_prev=True)
        return carry

    lax.fori_loop(1, MAIN_CHUNKS // NSLOT, loop_body, 0)

    for t in range(TAIL_CHUNKS):
        gather_wait(t)
        scatter_start(t, 2 * (MAIN_CHUNKS + t), owner(t % 2))
    for g in range(TAIL_CHUNKS, NSLOT - 1):
        gather_wait(g)
    scatter_drain(NSLOT - 1, owner((MAIN_CHUNKS - 1) % 2))
    for t in range(TAIL_CHUNKS):
        scatter_drain(t, owner(t % 2))

    @pl.when(s < EXTRA_CHUNKS)
    def _():
        k = NS * CHUNKS_PER_TILE + s
        pltpu.sync_copy(src(k), rows[NSLOT - 1])
        scatter_start(NSLOT - 1, TILE_IDXROWS, c == (s % 2))
        scatter_drain(NSLOT - 1, c == (s % 2))

    plsc.subcore_barrier()
    pltpu.sync_copy(acc.at[pl.ds(base_n, ROWS_PER_TILE)],
                    esum_out.at[pl.ds(base_n, ROWS_PER_TILE), pl.ds(col0, DW)])
    pltpu.sync_copy(cacc.at[pl.ds(base_n, ROWS_PER_TILE)],
                    cc_v.at[pl.ds(0, ROWS_PER_TILE)])
    zeros16 = jnp.zeros((16,), jnp.int32)
    iota16 = lax.iota(jnp.int32, 16)
    for k in range(CGROUPS):
        vals = plsc.load_gather(cc_v, [iota16 + 16 * k, zeros16])
        cflat_v[pl.ds(16 * k, 16)] = vals
    pltpu.sync_copy(cflat_v.at[pl.ds(0, ROWS_PER_TILE)], cnt_out.at[c, s])


_sc_scatter = functools.partial(
    pl.kernel,
    out_type=(
        jax.ShapeDtypeStruct((N, D), jnp.float32),
        jax.ShapeDtypeStruct((NC, NS, ROWS_PER_TILE), jnp.float32),
    ),
    mesh=plsc.VectorSubcoreMesh(
        core_axis_name="c", subcore_axis_name="s",
        num_cores=NC, num_subcores=NS),
    scratch_types=[
        [pltpu.VMEM((CHUNK_EDGES, DW), jnp.float32) for _ in range(NSLOT)],
        pltpu.VMEM((TILE_IDXROWS + CHUNK_IDXROWS, 128), jnp.int32),
        pltpu.VMEM((128, CNTW), jnp.float32),
        pltpu.VMEM((CGROUPS * 16, CNTW), jnp.float32),
        pltpu.VMEM((CGROUPS * 16,), jnp.float32),
        pltpu.VMEM_SHARED((N, DW), jnp.float32),
        pltpu.VMEM_SHARED((N, CNTW), jnp.float32),
        [pltpu.SemaphoreType.DMA for _ in range(NSLOT)],
        [pltpu.SemaphoreType.DMA for _ in range(NSLOT)],
    ],
    compiler_params=pltpu.CompilerParams(use_tc_tiling_on_sc=False, needs_layout_passes=False),
)(_sc_body)


def _tcx_body(x_ref, batch_ref, accx_out, accn_out, acc_x, acc_n):
    j = pl.program_id(0)

    @pl.when(j == 0)
    def _():
        acc_x[...] = jnp.zeros((B, D), jnp.float32)
        acc_n[...] = jnp.zeros((B, D), jnp.float32)

    bb = batch_ref[0]
    ids = lax.broadcasted_iota(jnp.int32, (B, NODE_BLK), 0)
    m = (bb == ids).astype(jnp.float32)

    acc_x[...] += jnp.dot(m, x_ref[...], preferred_element_type=jnp.float32)
    acc_n[...] += jnp.sum(m, axis=1, keepdims=True)

    @pl.when(j == NB - 1)
    def _():
        accx_out[...] = acc_x[...]
        accn_out[...] = acc_n[...]


def _tcx_call(x, batch3):
    res = lambda j: (0, 0)
    return pl.pallas_call(
        _tcx_body,
        grid=(NB,),
        in_specs=[
            pl.BlockSpec((NODE_BLK, D), lambda j: (j, 0)),
            pl.BlockSpec((1, 1, NODE_BLK), lambda j: (j, 0, 0)),
        ],
        out_specs=[pl.BlockSpec((B, D), res), pl.BlockSpec((B, D), res)],
        out_shape=[jax.ShapeDtypeStruct((B, D), jnp.float32),
                   jax.ShapeDtypeStruct((B, D), jnp.float32)],
        scratch_shapes=[pltpu.VMEM((B, D), jnp.float32)] * 2,
        compiler_params=pltpu.CompilerParams(
            dimension_semantics=("arbitrary",)),
    )(x, batch3)


def _tce_body(esum_hbm, cnt_ref, batch_ref, accx_ref, accn_ref, u_ref,
              w0_ref, w1_ref, w2_ref, b0_ref, b1_ref, b2_ref,
              out_ref, acc_e, ebuf, esem):
    j = pl.program_id(0)

    @pl.when(j == 0)
    def _():
        acc_e[...] = jnp.zeros((B, D), jnp.float32)
        pltpu.async_copy(esum_hbm, ebuf, esem).wait()

    bb = batch_ref[0]
    ids = lax.broadcasted_iota(jnp.int32, (B, NODE_BLK), 0)
    m = (bb == ids).astype(jnp.float32)

    deg = cnt_ref[0, 0] + cnt_ref[1, 0]
    minv = m / jnp.maximum(deg, 1.0)

    row0 = pl.multiple_of(j * NODE_BLK, 8)
    e = ebuf[pl.ds(row0, NODE_BLK), :]
    acc_e[...] += jnp.dot(minv, e, preferred_element_type=jnp.float32)

    @pl.when(j == NB - 1)
    def _():
        n = jnp.maximum(accn_ref[...], 1.0)
        u_e = acc_e[...] / n
        u_v = accx_ref[...] / n
        comb = jnp.concatenate([u_e, u_v, u_ref[...]], axis=1)
        dn = (((1,), (1,)), ((), ()))
        h = jnp.maximum(lax.dot_general(
            comb, w0_ref[...], dn, preferred_element_type=jnp.float32)
            + b0_ref[...], 0.0)
        h = jnp.maximum(lax.dot_general(
            h, w1_ref[...], dn, preferred_element_type=jnp.float32)
            + b1_ref[...], 0.0)
        h = jnp.maximum(lax.dot_general(
            h, w2_ref[...], dn, preferred_element_type=jnp.float32)
            + b2_ref[...], 0.0)
        out_ref[...] = h


def _tce_call(esum, cnt, batch3, accx, accn, u, W0, W1, W2, b0r, b1r, b2r):
    res = lambda j: (0, 0)
    return pl.pallas_call(
        _tce_body,
        grid=(NB,),
        in_specs=[
            pl.BlockSpec(memory_space=pl.ANY),
            pl.BlockSpec((NC, 1, 1, NODE_BLK), lambda j: (0, j, 0, 0)),
            pl.BlockSpec((1, 1, NODE_BLK), lambda j: (j, 0, 0)),
            pl.BlockSpec((B, D), res),
            pl.BlockSpec((B, D), res),
            pl.BlockSpec((B, D), res),
            pl.BlockSpec((D, 3 * D), res),
            pl.BlockSpec((D, D), res),
            pl.BlockSpec((D, D), res),
            pl.BlockSpec((1, D), res),
            pl.BlockSpec((1, D), res),
            pl.BlockSpec((1, D), res),
        ],
        out_specs=pl.BlockSpec((B, D), res),
        out_shape=jax.ShapeDtypeStruct((B, D), jnp.float32),
        scratch_shapes=[pltpu.VMEM((B, D), jnp.float32),
                        pltpu.VMEM((N, D), jnp.float32),
                        pltpu.SemaphoreType.DMA],
        compiler_params=pltpu.CompilerParams(
            dimension_semantics=("arbitrary",)),
    )(esum, cnt, batch3, accx, accn, u, W0, W1, W2, b0r, b1r, b2r)


def kernel(x, edge_index, edge_attr, u, batch, W0, b0, W1, b1, W2, b2):
    assert x.shape == (N, D) and edge_attr.shape == (E, D)
    assert u.shape == (B, D) and batch.shape == (N,)

    src = edge_index[0].reshape(IDXROWS, 128)
    zrow = jnp.zeros((ROWS_PER_TILE, DW), jnp.float32)
    zcnt = jnp.zeros((ROWS_PER_TILE, CNTW), jnp.float32)
    ones16 = jnp.ones((128, CNTW), jnp.float32)

    esum, cnt = _sc_scatter(edge_attr, src, zrow, zcnt, ones16)

    batch3 = batch.reshape(NB, 1, NODE_BLK)
    accx, accn = _tcx_call(x, batch3)
    cnt4 = cnt.reshape(NC, NB, 1, NODE_BLK)
    return _tce_call(esum, cnt4, batch3, accx, accn, u, W0, W1, W2,
                     b0.reshape(1, D), b1.reshape(1, D), b2.reshape(1, D))
